# baseline (speedup 1.0000x reference)
import functools

import jax
import jax.numpy as jnp
from jax import lax
from jax.experimental import pallas as pl
from jax.experimental.pallas import tpu as pltpu
from jax.experimental.pallas import tpu_sc as plsc

NUM_RESERVOIRS = 8192
EMBEDDING_DIM = 256
NUM_IDS = 262144

_info = plsc.get_sparse_core_info()
_NC = _info.num_cores
_NS = _info.num_subcores
_NW = _NC * _NS
_B_PER_W = NUM_IDS // _NW
_CHUNK = 32
_N_CHUNKS = _B_PER_W // _CHUNK
_NBUF = 8
_D = 4

_mesh = plsc.VectorSubcoreMesh(core_axis_name="c", subcore_axis_name="s")


@functools.partial(
    pl.kernel,
    mesh=_mesh,
    out_type=jax.ShapeDtypeStruct((NUM_IDS, EMBEDDING_DIM), jnp.float32),
    scratch_types=[
        pltpu.VMEM((_B_PER_W,), jnp.int32),
    ] + [pltpu.VMEM((_CHUNK, EMBEDDING_DIM), jnp.float32)] * _NBUF
      + [pltpu.SemaphoreType.DMA] * (2 * _NBUF),
)
def _gather_sc(table_hbm, idx_hbm, out_hbm, idx_v,
               r0, r1, r2, r3, r4, r5, r6, r7,
               g0, g1, g2, g3, g4, g5, g6, g7,
               o0, o1, o2, o3, o4, o5, o6, o7):
    rows = (r0, r1, r2, r3, r4, r5, r6, r7)
    gsem = (g0, g1, g2, g3, g4, g5, g6, g7)
    osem = (o0, o1, o2, o3, o4, o5, o6, o7)
    wid = lax.axis_index("s") * _NC + lax.axis_index("c")
    base = wid * _B_PER_W
    pltpu.sync_copy(idx_hbm.at[pl.ds(base, _B_PER_W)], idx_v)

    def start_gather(c, b):
        pltpu.async_copy(
            table_hbm.at[idx_v.at[pl.ds(c * _CHUNK, _CHUNK)]], rows[b], gsem[b])

    def wait_gather(b):
        pltpu.make_async_copy(
            table_hbm.at[pl.ds(0, _CHUNK)], rows[b], gsem[b]).wait()

    def start_out(c, b):
        pltpu.async_copy(
            rows[b], out_hbm.at[pl.ds(base + c * _CHUNK, _CHUNK)], osem[b])

    def wait_out(b):
        pltpu.make_async_copy(
            rows[b], out_hbm.at[pl.ds(base, _CHUNK)], osem[b]).wait()

    for b in range(_D):
        start_gather(b, b)
    for c in range(_D):
        start_gather(c + _D, (c + _D) % _NBUF)
        wait_gather(c % _NBUF)
        start_out(c, c % _NBUF)

    def outer(i, carry):
        for j in range(_NBUF):
            c = i * _NBUF + _D + j
            b = (_D + j) % _NBUF
            wait_out(j)
            start_gather(c + _D, j)
            wait_gather(b)
            start_out(c, b)
        return carry

    lax.fori_loop(0, (_N_CHUNKS - 2 * _D) // _NBUF, outer, 0)

    for k in range(_D):
        c = _N_CHUNKS - _D + k
        wait_out((c - _D) % _NBUF)
        wait_gather(c % _NBUF)
        start_out(c, c % _NBUF)
    for k in range(_D):
        wait_out((_N_CHUNKS - _D + k) % _NBUF)


def kernel(prototypes, reservoir_ids):
    idx = reservoir_ids.astype(jnp.int32)
    return _gather_sc(prototypes, idx)
